# Initial kernel scaffold; baseline (speedup 1.0000x reference)
import jax
import jax.numpy as jnp
from jax.experimental import pallas as pl


def kernel(x, edge_index, W1, b1, g1, bt1, W2, b2, g2, bt2):
    raise NotImplementedError("write your pallas kernel here")


# speedup vs baseline: 7.9663x; 7.9663x over previous
import functools

import jax
import jax.numpy as jnp
from jax import lax
from jax.experimental import pallas as pl
from jax.experimental.pallas import tpu as pltpu
from jax.experimental.pallas import tpu_sc as plsc

EPS = 1e-5

NT = 16
NC = 2
LANES = 128


def _sc_mesh():
    return plsc.VectorSubcoreMesh(core_axis_name="c", subcore_axis_name="s")


def _deg_kernel(edst, npad, ch):
    rpt = npad // NT

    @functools.partial(
        pl.kernel,
        mesh=_sc_mesh(),
        out_type=jax.ShapeDtypeStruct((npad, 16), jnp.float32),
        scratch_types=[
            pltpu.VMEM((ch, LANES), jnp.int32),
            pltpu.VMEM((LANES, 16), jnp.float32),
            pltpu.VMEM_SHARED((npad, 16), jnp.float32),
        ],
    )
    def run(edst_hbm, out_hbm, didx, ones_buf, dacc):
        c = lax.axis_index("c")
        s = lax.axis_index("s")
        pltpu.sync_copy(edst_hbm.at[s], didx)
        one16 = jnp.ones((16,), jnp.float32)

        @pl.loop(0, LANES)
        def _(rr):
            ones_buf[rr, :] = one16

        base = s * rpt
        nfull = rpt // LANES
        rem = rpt - nfull * LANES
        for k in range(nfull):
            pltpu.sync_copy(ones_buf, dacc.at[pl.ds(base + k * LANES, LANES)])
        if rem:
            pltpu.sync_copy(ones_buf.at[pl.ds(0, rem)],
                            dacc.at[pl.ds(base + nfull * LANES, rem)])
        plsc.subcore_barrier()

        @pl.loop(0, ch)
        def _(j):
            pltpu.sync_copy(ones_buf, dacc.at[didx.at[j]], add=True)

        plsc.subcore_barrier()

        @pl.when(c == 0)
        def _():
            pltpu.sync_copy(dacc.at[pl.ds(s * rpt, rpt)],
                            out_hbm.at[pl.ds(s * rpt, rpt)])

    return run(edst)


def _agg_kernel(h_split, esrc2, edst, npad, ch):
    rpt = npad // NT

    @functools.partial(
        pl.kernel,
        mesh=_sc_mesh(),
        out_type=jax.ShapeDtypeStruct((NC * npad, 128), jnp.float32),
        scratch_types=[
            pltpu.VMEM((ch, LANES), jnp.int32),
            pltpu.VMEM((ch, LANES), jnp.int32),
            pltpu.VMEM((LANES, 128), jnp.float32),
            pltpu.VMEM_SHARED((npad, 128), jnp.float32),
        ],
    )
    def run(h_hbm, esrc_hbm, edst_hbm, out_hbm, sidx, didx, buf, acc):
        c = lax.axis_index("c")
        s = lax.axis_index("s")
        pltpu.sync_copy(esrc_hbm.at[c, s], sidx)
        pltpu.sync_copy(edst_hbm.at[s], didx)

        zero16 = jnp.zeros((16,), jnp.float32)

        @pl.loop(0, LANES)
        def _(rr):
            for k in range(8):
                buf[rr, pl.ds(k * 16, 16)] = zero16

        base = s * rpt
        nfull = rpt // LANES
        rem = rpt - nfull * LANES
        for k in range(nfull):
            pltpu.sync_copy(buf, acc.at[pl.ds(base + k * LANES, LANES)])
        if rem:
            pltpu.sync_copy(buf.at[pl.ds(0, rem)],
                            acc.at[pl.ds(base + nfull * LANES, rem)])
        plsc.subcore_barrier()

        @pl.loop(0, ch)
        def _(j):
            pltpu.sync_copy(h_hbm.at[sidx.at[j]], buf)
            pltpu.sync_copy(buf, acc.at[didx.at[j]], add=True)

        plsc.subcore_barrier()
        pltpu.sync_copy(acc.at[pl.ds(s * rpt, rpt)],
                        out_hbm.at[pl.ds(c * npad + s * rpt, rpt)])

    return run(h_split, esrc2, edst)


def _mm_scale(x, w, deg16, n, npad, br):
    nb = n // br
    d = x.shape[1]

    def body(x_ref, w_ref, deg_ref, out_ref):
        dinv = lax.rsqrt(deg_ref[:, 0:1])
        out_ref[0] = jnp.dot(x_ref[...], w_ref[...],
                             preferred_element_type=jnp.float32) * dinv

    return pl.pallas_call(
        body,
        grid=(NC, nb),
        in_specs=[
            pl.BlockSpec((br, d), lambda c, r: (r, 0)),
            pl.BlockSpec((d, 128), lambda c, r: (0, c)),
            pl.BlockSpec((br, 16), lambda c, r: (r, 0)),
        ],
        out_specs=pl.BlockSpec((1, br, 128), lambda c, r: (c, r, 0)),
        out_shape=jax.ShapeDtypeStruct((NC, npad, 128), jnp.float32),
    )(x, w, deg16)


def _post_stats(a, h, deg16, n, npad, br):
    nb = n // br

    def body(a_ref, h_ref, deg_ref, p_ref, st_ref):
        r = pl.program_id(1)
        dinv = lax.rsqrt(deg_ref[:, 0:1])
        p = (a_ref[0] + h_ref[0]) * dinv
        p_ref[0] = p

        @pl.when(r == 0)
        def _():
            st_ref[...] = jnp.zeros_like(st_ref)

        st_ref[0, 0:1, :] += jnp.sum(p, axis=0, keepdims=True)
        st_ref[0, 1:2, :] += jnp.sum(p * p, axis=0, keepdims=True)

    return pl.pallas_call(
        body,
        grid=(NC, nb),
        in_specs=[
            pl.BlockSpec((1, br, 128), lambda c, r: (c, r, 0)),
            pl.BlockSpec((1, br, 128), lambda c, r: (c, r, 0)),
            pl.BlockSpec((br, 16), lambda c, r: (r, 0)),
        ],
        out_specs=[
            pl.BlockSpec((1, br, 128), lambda c, r: (c, r, 0)),
            pl.BlockSpec((1, 2, 128), lambda c, r: (c, 0, 0)),
        ],
        out_shape=[
            jax.ShapeDtypeStruct((NC, npad, 128), jnp.float32),
            jax.ShapeDtypeStruct((NC, 2, 128), jnp.float32),
        ],
    )(a, h, deg16)


def _bn_relu_halves(p0_ref, p1_ref, st_ref, g_ref, bt_ref, n, relu=True):
    zs = []
    for i, pref in ((0, p0_ref), (1, p1_ref)):
        m = st_ref[i, 0:1, :] * (1.0 / n)
        ex2 = st_ref[i, 1:2, :] * (1.0 / n)
        var = ex2 - m * m
        rstd = lax.rsqrt(var + EPS)
        z = (pref[0] - m) * (rstd * g_ref[i:i + 1, :]) + bt_ref[i:i + 1, :]
        zs.append(jnp.maximum(z, 0.0) if relu else z)
    return jnp.concatenate(zs, axis=1)


def _bn_mm_scale(p, st, g, bt, w, deg16, n, npad, br):
    nb = n // br
    d = w.shape[0]

    def body(p0_ref, p1_ref, st_ref, g_ref, bt_ref, w_ref, deg_ref, out_ref):
        z = _bn_relu_halves(p0_ref, p1_ref, st_ref, g_ref, bt_ref, n)
        dinv = lax.rsqrt(deg_ref[:, 0:1])
        out_ref[0] = jnp.dot(z, w_ref[...],
                             preferred_element_type=jnp.float32) * dinv

    return pl.pallas_call(
        body,
        grid=(NC, nb),
        in_specs=[
            pl.BlockSpec((1, br, 128), lambda c, r: (0, r, 0)),
            pl.BlockSpec((1, br, 128), lambda c, r: (1, r, 0)),
            pl.BlockSpec((NC, 2, 128), lambda c, r: (0, 0, 0)),
            pl.BlockSpec((NC, 128), lambda c, r: (0, 0)),
            pl.BlockSpec((NC, 128), lambda c, r: (0, 0)),
            pl.BlockSpec((d, 128), lambda c, r: (0, c)),
            pl.BlockSpec((br, 16), lambda c, r: (r, 0)),
        ],
        out_specs=pl.BlockSpec((1, br, 128), lambda c, r: (c, r, 0)),
        out_shape=jax.ShapeDtypeStruct((NC, npad, 128), jnp.float32),
    )(p, p, st, g, bt, w, deg16)


def _bn_residual(p, st, g, bt, x, n, npad, br):
    nb = n // br
    d = x.shape[1]

    def body(p0_ref, p1_ref, st_ref, g_ref, bt_ref, x_ref, out_ref):
        z = _bn_relu_halves(p0_ref, p1_ref, st_ref, g_ref, bt_ref, n,
                            relu=False)
        out_ref[...] = jnp.maximum(z + x_ref[...], 0.0)

    return pl.pallas_call(
        body,
        grid=(nb,),
        in_specs=[
            pl.BlockSpec((1, br, 128), lambda r: (0, r, 0)),
            pl.BlockSpec((1, br, 128), lambda r: (1, r, 0)),
            pl.BlockSpec((NC, 2, 128), lambda r: (0, 0, 0)),
            pl.BlockSpec((NC, 128), lambda r: (0, 0)),
            pl.BlockSpec((NC, 128), lambda r: (0, 0)),
            pl.BlockSpec((br, d), lambda r: (r, 0)),
        ],
        out_specs=pl.BlockSpec((br, d), lambda r: (r, 0)),
        out_shape=jax.ShapeDtypeStruct((n, d), jnp.float32),
    )(p, p, st, g, bt, x)


def kernel(x, edge_index, W1, b1, g1, bt1, W2, b2, g2, bt2):
    n, d = x.shape
    e = edge_index.shape[1]
    del b1, b2

    br = 400
    npad = ((n // 128) + 1) * 128
    ch = (((e + NT - 1) // NT) + LANES - 1) // LANES
    ept = ch * LANES
    ep = NT * ept

    src = edge_index[0].astype(jnp.int32)
    dst = edge_index[1].astype(jnp.int32)
    pad = ep - e
    src_p = jnp.concatenate([src, jnp.zeros((pad,), jnp.int32)])
    dst_p = jnp.concatenate([dst, jnp.full((pad,), n, jnp.int32)])
    esrc = src_p.reshape(NT, ch, LANES)
    esrc2 = jnp.stack([esrc, esrc + npad])
    edst = dst_p.reshape(NT, ch, LANES)

    g1r = g1.reshape(NC, 128)
    bt1r = bt1.reshape(NC, 128)
    g2r = g2.reshape(NC, 128)
    bt2r = bt2.reshape(NC, 128)

    deg16 = _deg_kernel(edst, npad, ch)

    h1 = _mm_scale(x, W1, deg16, n, npad, br)
    a1 = _agg_kernel(h1.reshape(NC * npad, 128), esrc2, edst, npad, ch)
    p1, st1 = _post_stats(a1.reshape(NC, npad, 128), h1, deg16, n, npad, br)

    h2 = _bn_mm_scale(p1, st1, g1r, bt1r, W2, deg16, n, npad, br)
    a2 = _agg_kernel(h2.reshape(NC * npad, 128), esrc2, edst, npad, ch)
    p2, st2 = _post_stats(a2.reshape(NC, npad, 128), h2, deg16, n, npad, br)

    return _bn_residual(p2, st2, g2r, bt2r, x, n, npad, br)
